# speedup vs baseline: 3.0896x; 3.0896x over previous
import functools

import jax
import jax.numpy as jnp
from jax import lax
from jax.experimental import pallas as pl
from jax.experimental.pallas import tpu as pltpu
from jax.experimental.pallas import tpu_sc as plsc

_L = 16


def kernel(features, embedding_list):
    B, N, d1 = features.shape
    _, R, d2 = embedding_list.shape
    D = d1 + d2
    NC, NS = 2, 16
    NW = NC * NS
    wpb = NW // B
    n_per_w = N // wpb

    mesh = plsc.VectorSubcoreMesh(
        core_axis_name="c", subcore_axis_name="s", num_cores=NC, num_subcores=NS
    )

    @functools.partial(
        pl.kernel,
        out_type=jax.ShapeDtypeStruct((B, N, R, D), jnp.float32),
        mesh=mesh,
        scratch_types=[
            pltpu.VMEM((R, D), jnp.float32),
            pltpu.VMEM((R, D), jnp.float32),
            pltpu.VMEM((n_per_w, d1), jnp.float32),
            pltpu.VMEM((R, d2), jnp.float32),
            pltpu.SemaphoreType.DMA,
            pltpu.SemaphoreType.DMA,
        ],
    )
    def merge(feat_hbm, emb_hbm, out_hbm, tile0, tile1, feat_v, emb_v, sem0, sem1):
        cid = lax.axis_index("c")
        sid = lax.axis_index("s")
        wid = sid * NC + cid
        b = wid // wpb
        i0 = (wid % wpb) * n_per_w

        pltpu.sync_copy(feat_hbm.at[b, pl.ds(i0, n_per_w)], feat_v)
        pltpu.sync_copy(emb_hbm.at[b], emb_v)

        def place(j, carry):
            for k in range(d2 // _L):
                v = emb_v[j, pl.ds(k * _L, _L)]
                tile0[j, pl.ds(d1 + k * _L, _L)] = v
                tile1[j, pl.ds(d1 + k * _L, _L)] = v
            return carry

        lax.fori_loop(0, R, place, 0)

        tiles = (tile0, tile1)
        sems = (sem0, sem1)

        def build(tile, i):
            vs = [feat_v[i, pl.ds(k * _L, _L)] for k in range(d1 // _L)]

            def row(j, carry):
                for k in range(d1 // _L):
                    tile[j, pl.ds(k * _L, _L)] = vs[k]
                return carry

            lax.fori_loop(0, R, row, 0)

        for i in range(n_per_w):
            buf = i % 2
            if i >= 2:
                pltpu.make_async_copy(
                    tiles[buf], out_hbm.at[b, i0 + (i - 2)], sems[buf]
                ).wait()
            build(tiles[buf], i)
            pltpu.async_copy(tiles[buf], out_hbm.at[b, i0 + i], sems[buf])

        for i in (n_per_w - 2, n_per_w - 1):
            pltpu.make_async_copy(
                tiles[i % 2], out_hbm.at[b, i0 + i], sems[i % 2]
            ).wait()

    return merge(features, embedding_list)


# speedup vs baseline: 3.1926x; 1.0334x over previous
import functools

import jax
import jax.numpy as jnp
from jax import lax
from jax.experimental import pallas as pl
from jax.experimental.pallas import tpu as pltpu
from jax.experimental.pallas import tpu_sc as plsc

_L = 16


def kernel(features, embedding_list):
    B, N, d1 = features.shape
    _, R, d2 = embedding_list.shape
    D = d1 + d2
    NC, NS = 2, 16
    NW = NC * NS
    wpb = NW // B
    n_per_w = N // wpb

    mesh = plsc.VectorSubcoreMesh(
        core_axis_name="c", subcore_axis_name="s", num_cores=NC, num_subcores=NS
    )

    @functools.partial(
        pl.kernel,
        out_type=jax.ShapeDtypeStruct((B, N, R, D), jnp.float32),
        mesh=mesh,
        scratch_types=[
            pltpu.VMEM((R, D), jnp.float32),
            pltpu.VMEM((R, D), jnp.float32),
            pltpu.VMEM((n_per_w, d1), jnp.float32),
            pltpu.VMEM((R, d2), jnp.float32),
            pltpu.SemaphoreType.DMA,
            pltpu.SemaphoreType.DMA,
        ],
    )
    def merge(feat_hbm, emb_hbm, out_hbm, tile0, tile1, feat_v, emb_v, sem0, sem1):
        cid = lax.axis_index("c")
        sid = lax.axis_index("s")
        wid = sid * NC + cid
        b = wid // wpb
        i0 = (wid % wpb) * n_per_w

        pltpu.sync_copy(feat_hbm.at[b, pl.ds(i0, n_per_w)], feat_v)
        pltpu.sync_copy(emb_hbm.at[b], emb_v)

        tiles = (tile0, tile1)
        sems = (sem0, sem1)

        def place(tile):
            @functools.partial(plsc.parallel_loop, 0, R, unroll=4)
            def _(j):
                for k in range(d2 // _L):
                    tile[j, pl.ds(d1 + k * _L, _L)] = emb_v[j, pl.ds(k * _L, _L)]

        def build(tile, i):
            vs = [feat_v[i, pl.ds(k * _L, _L)] for k in range(d1 // _L)]

            @functools.partial(plsc.parallel_loop, 0, R, unroll=8)
            def _(j):
                for k in range(d1 // _L):
                    tile[j, pl.ds(k * _L, _L)] = vs[k]

        for i in range(n_per_w):
            buf = i % 2
            if i < 2:
                place(tiles[buf])
            else:
                pltpu.make_async_copy(
                    tiles[buf], out_hbm.at[b, i0 + (i - 2)], sems[buf]
                ).wait()
            build(tiles[buf], i)
            pltpu.async_copy(tiles[buf], out_hbm.at[b, i0 + i], sems[buf])

        for i in (n_per_w - 2, n_per_w - 1):
            pltpu.make_async_copy(
                tiles[i % 2], out_hbm.at[b, i0 + i], sems[i % 2]
            ).wait()

    return merge(features, embedding_list)


# speedup vs baseline: 3.2166x; 1.0075x over previous
import functools

import jax
import jax.numpy as jnp
from jax import lax
from jax.experimental import pallas as pl
from jax.experimental.pallas import tpu as pltpu
from jax.experimental.pallas import tpu_sc as plsc

_L = 16


def kernel(features, embedding_list):
    B, N, d1 = features.shape
    _, R, d2 = embedding_list.shape
    D = d1 + d2
    NC, NS = 2, 16
    NW = NC * NS
    wpb = NW // B
    n_per_w = N // wpb

    mesh = plsc.VectorSubcoreMesh(
        core_axis_name="c", subcore_axis_name="s", num_cores=NC, num_subcores=NS
    )

    @functools.partial(
        pl.kernel,
        out_type=jax.ShapeDtypeStruct((B, N, R, D), jnp.float32),
        mesh=mesh,
        scratch_types=[
            pltpu.VMEM((R, D), jnp.float32),
            pltpu.VMEM((R, D), jnp.float32),
            pltpu.VMEM((n_per_w, d1), jnp.float32),
            pltpu.VMEM((R, d2), jnp.float32),
            pltpu.SemaphoreType.DMA,
            pltpu.SemaphoreType.DMA,
        ],
    )
    def merge(feat_hbm, emb_hbm, out_hbm, tile0, tile1, feat_v, emb_v, sem0, sem1):
        cid = lax.axis_index("c")
        sid = lax.axis_index("s")
        wid = sid * NC + cid
        b = wid // wpb
        i0 = (wid % wpb) * n_per_w

        tiles = (tile0, tile1)
        sems = (sem0, sem1)

        pltpu.async_copy(feat_hbm.at[b, pl.ds(i0, n_per_w)], feat_v, sem0)
        pltpu.async_copy(emb_hbm.at[b], emb_v, sem1)
        pltpu.make_async_copy(feat_hbm.at[b, pl.ds(i0, n_per_w)], feat_v, sem0).wait()
        pltpu.make_async_copy(emb_hbm.at[b], emb_v, sem1).wait()

        def place(tile):
            @functools.partial(plsc.parallel_loop, 0, R, unroll=4)
            def _(j):
                for k in range(d2 // _L):
                    tile[j, pl.ds(d1 + k * _L, _L)] = emb_v[j, pl.ds(k * _L, _L)]

        def build(tile, i):
            vs = [feat_v[i, pl.ds(k * _L, _L)] for k in range(d1 // _L)]

            @functools.partial(plsc.parallel_loop, 0, R, unroll=8)
            def _(j):
                for k in range(d1 // _L):
                    tile[j, pl.ds(k * _L, _L)] = vs[k]

        for i in range(n_per_w):
            buf = i % 2
            if i < 2:
                place(tiles[buf])
            else:
                pltpu.make_async_copy(
                    tiles[buf], out_hbm.at[b, i0 + (i - 2)], sems[buf]
                ).wait()
            build(tiles[buf], i)
            pltpu.async_copy(tiles[buf], out_hbm.at[b, i0 + i], sems[buf])

        for i in (n_per_w - 2, n_per_w - 1):
            pltpu.make_async_copy(
                tiles[i % 2], out_hbm.at[b, i0 + i], sems[i % 2]
            ).wait()

    return merge(features, embedding_list)
